# baseline (speedup 1.0000x reference)
import functools
import math

import numpy as np
import jax
import jax.numpy as jnp
from jax import lax
from jax.experimental import pallas as pl
from jax.experimental.pallas import tpu as pltpu
from jax.experimental.pallas import tpu_sc as plsc

_FM_SIZES = [(64, 64), (32, 32), (16, 16), (8, 8), (4, 4), (2, 2)]
_STEPS = [(8, 8), (16, 16), (32, 32), (64, 64), (128, 128), (256, 256)]
_BOX_SIZES = [35.84, 76.8, 153.6, 230.4, 307.2, 384.0]
_ASPECT_RATIOS = [1.0, 2.0, 0.5]
_SCALES = [1.0, 1.2599]
_FG = 0.6
_BG = 0.4

_G = 64
_P = 32760
_PP = 32768

_NC = 2
_NS = 16
_NW = _NC * _NS
_L = 16
_NG = _G // _L
_PW = _PP // _NW
_NCHW = _PW // _L


def _priors_np():
    out = []
    for i, (fy, fx) in enumerate(_FM_SIZES):
        sy, sx = _STEPS[i]
        base = _BOX_SIZES[i]
        hh, ww = np.meshgrid(np.arange(fy), np.arange(fx), indexing="ij")
        cx = (ww + 0.5) * sx
        cy = (hh + 0.5) * sy
        cxy = np.stack([cx, cy], axis=-1).reshape(-1, 1, 2).astype(np.float32)
        whs = []
        for ar in _ASPECT_RATIOS:
            for sc in _SCALES:
                whs.append((base * sc * math.sqrt(ar), base * sc / math.sqrt(ar)))
        wh = np.asarray(whs, dtype=np.float32).reshape(1, -1, 2)
        a = wh.shape[1]
        ncell = cxy.shape[0]
        b = np.concatenate(
            [np.broadcast_to(cxy, (ncell, a, 2)), np.broadcast_to(wh, (ncell, a, 2))],
            axis=-1,
        )
        out.append(b.reshape(-1, 4).astype(np.float32))
    return np.concatenate(out, axis=0)


def _prior_consts():
    d = _priors_np()
    x1y1 = d[:, :2] - d[:, 2:] / 2.0
    x2y2 = d[:, :2] + d[:, 2:] / 2.0
    dxy = np.concatenate([x1y1, x2y2], axis=1).astype(np.float32)
    area = (dxy[:, 2] - dxy[:, 0]) * (dxy[:, 3] - dxy[:, 1])

    def pad(v, fill):
        o = np.full((_PP,), fill, np.float32)
        o[:_P] = v
        return o

    return {
        "px1": pad(dxy[:, 0], -1.0e6), "py1": pad(dxy[:, 1], -1.0e6),
        "px2": pad(dxy[:, 2], -1.0e6), "py2": pad(dxy[:, 3], -1.0e6),
        "pa": pad(area, 0.0),
        "pcx": pad(d[:, 0], 0.0), "pcy": pad(d[:, 1], 0.0),
        "pw": pad(d[:, 2], 1.0), "ph": pad(d[:, 3], 1.0),
        "plw": pad(np.log(d[:, 2]), 0.0), "plh": pad(np.log(d[:, 3]), 0.0),
    }


_PC = _prior_consts()

_MESH = plsc.VectorSubcoreMesh(core_axis_name="c", subcore_axis_name="s")


def _tree_max_f(v, buf):
    r = v
    for sh in (8, 4, 2, 1):
        buf[0:_L] = r
        r = jnp.maximum(r, buf[pl.ds(sh, _L)])
    return r[0]


def _tree_min_i(v, buf):
    r = v
    for sh in (8, 4, 2, 1):
        buf[0:_L] = r
        r = jnp.minimum(r, buf[pl.ds(sh, _L)])
    return r[0]


@functools.partial(
    pl.kernel,
    out_type=[
        jax.ShapeDtypeStruct((_PP,), jnp.float32),
        jax.ShapeDtypeStruct((_PP,), jnp.int32),
        jax.ShapeDtypeStruct((_NW * _G,), jnp.float32),
        jax.ShapeDtypeStruct((_NW * _G,), jnp.int32),
    ],
    mesh=_MESH,
    scratch_types=[
        pltpu.VMEM((_PW,), jnp.float32),
        pltpu.VMEM((_PW,), jnp.float32),
        pltpu.VMEM((_PW,), jnp.float32),
        pltpu.VMEM((_PW,), jnp.float32),
        pltpu.VMEM((_PW,), jnp.float32),
        pltpu.VMEM((_G,), jnp.float32),
        pltpu.VMEM((_G,), jnp.float32),
        pltpu.VMEM((_G,), jnp.float32),
        pltpu.VMEM((_G,), jnp.float32),
        pltpu.VMEM((_G,), jnp.float32),
        pltpu.VMEM((_PW,), jnp.float32),
        pltpu.VMEM((_PW,), jnp.int32),
        pltpu.VMEM((_G,), jnp.float32),
        pltpu.VMEM((_G,), jnp.int32),
        pltpu.VMEM((2 * _L,), jnp.float32),
        pltpu.VMEM((2 * _L,), jnp.int32),
    ],
)
def _sc_match(px1_h, py1_h, px2_h, py2_h, pa_h,
              gx1_h, gy1_h, gx2_h, gy2_h, ga_h,
              bt_h, bi_h, cv_h, ci_h,
              px1, py1, px2, py2, pa,
              gx1, gy1, gx2, gy2, ga,
              bt, bi, cv, ci, redf, redi):
    wid = lax.axis_index("s") * _NC + lax.axis_index("c")
    base = wid * _PW
    pltpu.sync_copy(px1_h.at[pl.ds(base, _PW)], px1)
    pltpu.sync_copy(py1_h.at[pl.ds(base, _PW)], py1)
    pltpu.sync_copy(px2_h.at[pl.ds(base, _PW)], px2)
    pltpu.sync_copy(py2_h.at[pl.ds(base, _PW)], py2)
    pltpu.sync_copy(pa_h.at[pl.ds(base, _PW)], pa)
    pltpu.sync_copy(gx1_h, gx1)
    pltpu.sync_copy(gy1_h, gy1)
    pltpu.sync_copy(gx2_h, gx2)
    pltpu.sync_copy(gy2_h, gy2)
    pltpu.sync_copy(ga_h, ga)

    lane = lax.iota(jnp.int32, _L)
    redf[_L:2 * _L] = jnp.full((_L,), -3.0e38, jnp.float32)
    redi[_L:2 * _L] = jnp.full((_L,), 1 << 30, jnp.int32)
    gvec = []
    for j in range(_NG):
        s = slice(j * _L, (j + 1) * _L)
        gvec.append((gx1[s], gy1[s], gx2[s], gy2[s], ga[s]))

    _B = 8
    cvacc = [jnp.full((_L,), -1.0, jnp.float32) for _ in range(_NG)]
    ciacc = [jnp.zeros((_L,), jnp.int32) for _ in range(_NG)]
    for jg in range(_G // _B):
        g0 = jg * _B

        def body(c, carry, _g0=g0):
            rvs = carry[:_B]
            rcs = carry[_B:]
            o = c * _L
            x1 = px1[pl.ds(o, _L)]
            y1 = py1[pl.ds(o, _L)]
            x2 = px2[pl.ds(o, _L)]
            y2 = py2[pl.ds(o, _L)]
            ar = pa[pl.ds(o, _L)]
            if _g0 == 0:
                btv = None
                biv = None
            else:
                btv = bt[pl.ds(o, _L)]
                biv = bi[pl.ds(o, _L)]
            nrv = []
            nrc = []
            for t in range(_B):
                g = _g0 + t
                j, k = divmod(g, _L)
                vx1, vy1, vx2, vy2, va = gvec[j]
                ltx = jnp.maximum(x1, vx1[k])
                lty = jnp.maximum(y1, vy1[k])
                rbx = jnp.minimum(x2, vx2[k])
                rby = jnp.minimum(y2, vy2[k])
                wx = jnp.maximum(rbx - ltx, 0.0)
                wy = jnp.maximum(rby - lty, 0.0)
                inter = wx * wy
                den = (ar + va[k]) - inter + 1e-10
                iou = inter / den
                if g == 0:
                    btv = iou
                    biv = jnp.zeros((_L,), jnp.int32)
                else:
                    up = iou > btv
                    btv = jnp.where(up, iou, btv)
                    biv = jnp.where(up, g, biv)
                up2 = iou > rvs[t]
                nrv.append(jnp.where(up2, iou, rvs[t]))
                nrc.append(jnp.where(up2, c, rcs[t]))
            bt[pl.ds(o, _L)] = btv
            bi[pl.ds(o, _L)] = biv
            return tuple(nrv) + tuple(nrc)

        init = (tuple(jnp.full((_L,), -1.0, jnp.float32) for _ in range(_B))
                + tuple(jnp.zeros((_L,), jnp.int32) for _ in range(_B)))
        out = lax.fori_loop(0, _NCHW, body, init)
        for t in range(_B):
            g = g0 + t
            j, k = divmod(g, _L)
            rv = out[t]
            rc = out[_B + t]
            gidx = base + rc * _L + lane
            m = _tree_max_f(rv, redf)
            cand = jnp.where(rv == m, gidx, _PP)
            i = _tree_min_i(cand, redi)
            cvacc[j] = jnp.where(lane == k, m, cvacc[j])
            ciacc[j] = jnp.where(lane == k, i, ciacc[j])

    for j in range(_NG):
        s = slice(j * _L, (j + 1) * _L)
        cv[s] = cvacc[j]
        ci[s] = ciacc[j]

    pltpu.sync_copy(bt, bt_h.at[pl.ds(base, _PW)])
    pltpu.sync_copy(bi, bi_h.at[pl.ds(base, _PW)])
    pltpu.sync_copy(cv, cv_h.at[pl.ds(wid * _G, _G)])
    pltpu.sync_copy(ci, ci_h.at[pl.ds(wid * _G, _G)])


@functools.partial(
    pl.kernel,
    out_type=[
        jax.ShapeDtypeStruct((_PP,), jnp.float32),
        jax.ShapeDtypeStruct((_PP,), jnp.float32),
        jax.ShapeDtypeStruct((_PP,), jnp.float32),
        jax.ShapeDtypeStruct((_PP,), jnp.float32),
        jax.ShapeDtypeStruct((_PP,), jnp.int32),
    ],
    mesh=_MESH,
    scratch_types=[
        pltpu.VMEM((_PW,), jnp.float32),
        pltpu.VMEM((_PW,), jnp.int32),
        pltpu.VMEM((_NW * _G,), jnp.float32),
        pltpu.VMEM((_NW * _G,), jnp.int32),
        pltpu.VMEM((_PW,), jnp.float32),
        pltpu.VMEM((_PW,), jnp.float32),
        pltpu.VMEM((_PW,), jnp.float32),
        pltpu.VMEM((_PW,), jnp.float32),
        pltpu.VMEM((_PW,), jnp.float32),
        pltpu.VMEM((_PW,), jnp.float32),
        pltpu.VMEM((_G,), jnp.float32),
        pltpu.VMEM((_G,), jnp.float32),
        pltpu.VMEM((_G,), jnp.float32),
        pltpu.VMEM((_G,), jnp.float32),
        pltpu.VMEM((_G,), jnp.int32),
        pltpu.VMEM((_PW,), jnp.float32),
        pltpu.VMEM((_PW,), jnp.float32),
        pltpu.VMEM((_PW,), jnp.float32),
        pltpu.VMEM((_PW,), jnp.float32),
        pltpu.VMEM((_PW,), jnp.int32),
    ],
)
def _sc_encode(bt_h, bi_h, cv_h, ci_h,
               pcx_h, pcy_h, pw_h, ph_h, plw_h, plh_h,
               bcx_h, bcy_h, lbw_h, lbh_h, glab_h,
               lx_h, ly_h, lw_h, lh_h, cls_h,
               bt, bi, cv, ci,
               pcx, pcy, pw, ph, plw, plh,
               bcx, bcy, lbw, lbh, glab,
               olx, oly, olw, olh, ocls):
    wid = lax.axis_index("s") * _NC + lax.axis_index("c")
    base = wid * _PW
    pltpu.sync_copy(bt_h.at[pl.ds(base, _PW)], bt)
    pltpu.sync_copy(bi_h.at[pl.ds(base, _PW)], bi)
    pltpu.sync_copy(cv_h, cv)
    pltpu.sync_copy(ci_h, ci)
    pltpu.sync_copy(pcx_h.at[pl.ds(base, _PW)], pcx)
    pltpu.sync_copy(pcy_h.at[pl.ds(base, _PW)], pcy)
    pltpu.sync_copy(pw_h.at[pl.ds(base, _PW)], pw)
    pltpu.sync_copy(ph_h.at[pl.ds(base, _PW)], ph)
    pltpu.sync_copy(plw_h.at[pl.ds(base, _PW)], plw)
    pltpu.sync_copy(plh_h.at[pl.ds(base, _PW)], plh)
    pltpu.sync_copy(bcx_h, bcx)
    pltpu.sync_copy(bcy_h, bcy)
    pltpu.sync_copy(lbw_h, lbw)
    pltpu.sync_copy(lbh_h, lbh)
    pltpu.sync_copy(glab_h, glab)

    lane = lax.iota(jnp.int32, _L)

    def wred(w, carry):
        rvs, ris = carry
        nv = []
        ni = []
        for j in range(_NG):
            v = cv[pl.ds(w * _G + j * _L, _L)]
            ii = ci[pl.ds(w * _G + j * _L, _L)]
            rv = rvs[j]
            ri = ris[j]
            gtm = v > rv
            nv.append(jnp.where(gtm, v, rv))
            ni.append(jnp.where(gtm, ii,
                                jnp.where(v == rv, jnp.minimum(ii, ri), ri)))
        return (tuple(nv), tuple(ni))

    rvs0 = tuple(jnp.full((_L,), -1.0, jnp.float32) for _ in range(_NG))
    ris0 = tuple(jnp.full((_L,), _PP, jnp.int32) for _ in range(_NG))
    _, ris = lax.fori_loop(0, _NW, wred, (rvs0, ris0))

    for g in range(_G):
        j, k = divmod(g, _L)
        i = ris[j][k]
        loc = i - base
        locc = jnp.clip(loc, 0, _PW - 1)
        cb = (locc // _L) * _L
        kk = locc - cb
        ksel = jnp.where(loc >= 0, jnp.where(loc < _PW, kk, -1), -1)
        row = bt[pl.ds(cb, _L)]
        rowi = bi[pl.ds(cb, _L)]
        sel = lane == ksel
        bt[pl.ds(cb, _L)] = jnp.where(sel, 2.0, row)
        bi[pl.ds(cb, _L)] = jnp.where(sel, g, rowi)

    tcx = [bcx[slice(j * _L, (j + 1) * _L)] for j in range(_NG)]
    tcy = [bcy[slice(j * _L, (j + 1) * _L)] for j in range(_NG)]
    tlw = [lbw[slice(j * _L, (j + 1) * _L)] for j in range(_NG)]
    tlh = [lbh[slice(j * _L, (j + 1) * _L)] for j in range(_NG)]
    tlb = [glab[slice(j * _L, (j + 1) * _L)] for j in range(_NG)]

    def encode(c, carry):
        o = c * _L
        btv = bt[pl.ds(o, _L)]
        biv = bi[pl.ds(o, _L)]
        gcx = jnp.full((_L,), tcx[0][0], jnp.float32)
        gcy = jnp.full((_L,), tcy[0][0], jnp.float32)
        glw = jnp.full((_L,), tlw[0][0], jnp.float32)
        glh = jnp.full((_L,), tlh[0][0], jnp.float32)
        lab = jnp.full((_L,), tlb[0][0], jnp.int32)
        for g in range(1, _G):
            j, k = divmod(g, _L)
            eq = biv == g
            gcx = jnp.where(eq, tcx[j][k], gcx)
            gcy = jnp.where(eq, tcy[j][k], gcy)
            glw = jnp.where(eq, tlw[j][k], glw)
            glh = jnp.where(eq, tlh[j][k], glh)
            lab = jnp.where(eq, tlb[j][k], lab)
        olx[pl.ds(o, _L)] = (gcx - pcx[pl.ds(o, _L)]) / pw[pl.ds(o, _L)] / 0.1
        oly[pl.ds(o, _L)] = (gcy - pcy[pl.ds(o, _L)]) / ph[pl.ds(o, _L)] / 0.1
        olw[pl.ds(o, _L)] = (glw - plw[pl.ds(o, _L)]) / 0.2
        olh[pl.ds(o, _L)] = (glh - plh[pl.ds(o, _L)]) / 0.2
        cls = jnp.where(btv < _FG, -1, lab)
        cls = jnp.where(btv < _BG, 0, cls)
        ocls[pl.ds(o, _L)] = cls
        return carry

    lax.fori_loop(0, _NCHW, encode, 0)

    pltpu.sync_copy(olx, lx_h.at[pl.ds(base, _PW)])
    pltpu.sync_copy(oly, ly_h.at[pl.ds(base, _PW)])
    pltpu.sync_copy(olw, lw_h.at[pl.ds(base, _PW)])
    pltpu.sync_copy(olh, lh_h.at[pl.ds(base, _PW)])
    pltpu.sync_copy(ocls, cls_h.at[pl.ds(base, _PW)])


def kernel(gt_boxes, labels):
    gt_boxes = gt_boxes.astype(jnp.float32)
    gx1 = gt_boxes[:, 0]
    gy1 = gt_boxes[:, 1]
    gx2 = gt_boxes[:, 2]
    gy2 = gt_boxes[:, 3]
    gar = (gx2 - gx1) * (gy2 - gy1)
    bcx = (gx1 + gx2) / 2.0
    bcy = (gy1 + gy2) / 2.0
    lbw = jnp.log(gx2 - gx1)
    lbh = jnp.log(gy2 - gy1)
    glab = (labels + 1).astype(jnp.int32)

    c = {k: jnp.asarray(v) for k, v in _PC.items()}
    bt, bi, cv, ci = _sc_match(
        c["px1"], c["py1"], c["px2"], c["py2"], c["pa"],
        gx1, gy1, gx2, gy2, gar)
    lx, ly, lw, lh, cls = _sc_encode(
        bt, bi, cv, ci,
        c["pcx"], c["pcy"], c["pw"], c["ph"], c["plw"], c["plh"],
        bcx, bcy, lbw, lbh, glab)
    loc = jnp.stack([lx, ly, lw, lh], axis=1)[:_P]
    return (loc, cls[:_P])


# speedup vs baseline: 1.0660x; 1.0660x over previous
import functools
import math

import numpy as np
import jax
import jax.numpy as jnp
from jax import lax
from jax.experimental import pallas as pl
from jax.experimental.pallas import tpu as pltpu
from jax.experimental.pallas import tpu_sc as plsc

_FM_SIZES = [(64, 64), (32, 32), (16, 16), (8, 8), (4, 4), (2, 2)]
_STEPS = [(8, 8), (16, 16), (32, 32), (64, 64), (128, 128), (256, 256)]
_BOX_SIZES = [35.84, 76.8, 153.6, 230.4, 307.2, 384.0]
_ASPECT_RATIOS = [1.0, 2.0, 0.5]
_SCALES = [1.0, 1.2599]
_FG = 0.6
_BG = 0.4

_G = 64
_P = 32760
_PP = 32768

_NC = 2
_NS = 16
_NW = _NC * _NS
_L = 16
_NG = _G // _L
_PW = _PP // _NW
_NCHW = _PW // _L


def _priors_np():
    out = []
    for i, (fy, fx) in enumerate(_FM_SIZES):
        sy, sx = _STEPS[i]
        base = _BOX_SIZES[i]
        hh, ww = np.meshgrid(np.arange(fy), np.arange(fx), indexing="ij")
        cx = (ww + 0.5) * sx
        cy = (hh + 0.5) * sy
        cxy = np.stack([cx, cy], axis=-1).reshape(-1, 1, 2).astype(np.float32)
        whs = []
        for ar in _ASPECT_RATIOS:
            for sc in _SCALES:
                whs.append((base * sc * math.sqrt(ar), base * sc / math.sqrt(ar)))
        wh = np.asarray(whs, dtype=np.float32).reshape(1, -1, 2)
        a = wh.shape[1]
        ncell = cxy.shape[0]
        b = np.concatenate(
            [np.broadcast_to(cxy, (ncell, a, 2)), np.broadcast_to(wh, (ncell, a, 2))],
            axis=-1,
        )
        out.append(b.reshape(-1, 4).astype(np.float32))
    return np.concatenate(out, axis=0)


def _prior_consts():
    d = _priors_np()
    x1y1 = d[:, :2] - d[:, 2:] / 2.0
    x2y2 = d[:, :2] + d[:, 2:] / 2.0
    dxy = np.concatenate([x1y1, x2y2], axis=1).astype(np.float32)
    area = (dxy[:, 2] - dxy[:, 0]) * (dxy[:, 3] - dxy[:, 1])

    def pad(v, fill):
        o = np.full((_PP,), fill, np.float32)
        o[:_P] = v
        return o

    return {
        "px1": pad(dxy[:, 0], -1.0e6), "py1": pad(dxy[:, 1], -1.0e6),
        "px2": pad(dxy[:, 2], -1.0e6), "py2": pad(dxy[:, 3], -1.0e6),
        "pa": pad(area, 0.0),
        "pcx": pad(d[:, 0], 0.0), "pcy": pad(d[:, 1], 0.0),
        "pw": pad(d[:, 2], 1.0), "ph": pad(d[:, 3], 1.0),
        "plw": pad(np.log(d[:, 2]), 0.0), "plh": pad(np.log(d[:, 3]), 0.0),
    }


_PC = _prior_consts()

_MESH = plsc.VectorSubcoreMesh(core_axis_name="c", subcore_axis_name="s")


def _tree_max_f(v, buf):
    r = v
    for sh in (8, 4, 2, 1):
        buf[0:_L] = r
        r = jnp.maximum(r, buf[pl.ds(sh, _L)])
    return r[0]


def _tree_min_i(v, buf):
    r = v
    for sh in (8, 4, 2, 1):
        buf[0:_L] = r
        r = jnp.minimum(r, buf[pl.ds(sh, _L)])
    return r[0]


@functools.partial(
    pl.kernel,
    out_type=[
        jax.ShapeDtypeStruct((_PP,), jnp.float32),
        jax.ShapeDtypeStruct((_PP,), jnp.int32),
        jax.ShapeDtypeStruct((_NW * _G,), jnp.float32),
        jax.ShapeDtypeStruct((_NW * _G,), jnp.int32),
    ],
    mesh=_MESH,
    scratch_types=[
        pltpu.VMEM((_PW,), jnp.float32),
        pltpu.VMEM((_PW,), jnp.float32),
        pltpu.VMEM((_PW,), jnp.float32),
        pltpu.VMEM((_PW,), jnp.float32),
        pltpu.VMEM((_PW,), jnp.float32),
        pltpu.VMEM((_G,), jnp.float32),
        pltpu.VMEM((_G,), jnp.float32),
        pltpu.VMEM((_G,), jnp.float32),
        pltpu.VMEM((_G,), jnp.float32),
        pltpu.VMEM((_G,), jnp.float32),
        pltpu.VMEM((_PW,), jnp.float32),
        pltpu.VMEM((_PW,), jnp.int32),
        pltpu.VMEM((_G,), jnp.float32),
        pltpu.VMEM((_G,), jnp.int32),
        pltpu.VMEM((2 * _L,), jnp.float32),
        pltpu.VMEM((2 * _L,), jnp.int32),
        pltpu.VMEM((_G, _L), jnp.float32),
        pltpu.VMEM((_G, _L), jnp.int32),
    ],
)
def _sc_match(px1_h, py1_h, px2_h, py2_h, pa_h,
              gx1_h, gy1_h, gx2_h, gy2_h, ga_h,
              bt_h, bi_h, cv_h, ci_h,
              px1, py1, px2, py2, pa,
              gx1, gy1, gx2, gy2, ga,
              bt, bi, cv, ci, redf, redi, gbv, gbc):
    wid = lax.axis_index("s") * _NC + lax.axis_index("c")
    base = wid * _PW
    pltpu.sync_copy(px1_h.at[pl.ds(base, _PW)], px1)
    pltpu.sync_copy(py1_h.at[pl.ds(base, _PW)], py1)
    pltpu.sync_copy(px2_h.at[pl.ds(base, _PW)], px2)
    pltpu.sync_copy(py2_h.at[pl.ds(base, _PW)], py2)
    pltpu.sync_copy(pa_h.at[pl.ds(base, _PW)], pa)
    pltpu.sync_copy(gx1_h, gx1)
    pltpu.sync_copy(gy1_h, gy1)
    pltpu.sync_copy(gx2_h, gx2)
    pltpu.sync_copy(gy2_h, gy2)
    pltpu.sync_copy(ga_h, ga)

    lane = lax.iota(jnp.int32, _L)
    redf[_L:2 * _L] = jnp.full((_L,), -3.0e38, jnp.float32)
    redi[_L:2 * _L] = jnp.full((_L,), 1 << 30, jnp.int32)
    gvec = []
    for j in range(_NG):
        s = slice(j * _L, (j + 1) * _L)
        gvec.append((gx1[s], gy1[s], gx2[s], gy2[s], ga[s]))

    for g in range(_G):
        gbv[g] = jnp.full((_L,), -1.0, jnp.float32)
        gbc[g] = jnp.zeros((_L,), jnp.int32)

    def body(cc, carry):
        for half in range(2):
            c = cc * 2 + half
            o = c * _L
            x1 = px1[pl.ds(o, _L)]
            y1 = py1[pl.ds(o, _L)]
            x2 = px2[pl.ds(o, _L)]
            y2 = py2[pl.ds(o, _L)]
            ar = pa[pl.ds(o, _L)]
            btc = None
            bic = None
            for g in range(_G):
                j, k = divmod(g, _L)
                vx1, vy1, vx2, vy2, va = gvec[j]
                ltx = jnp.maximum(x1, vx1[k])
                lty = jnp.maximum(y1, vy1[k])
                rbx = jnp.minimum(x2, vx2[k])
                rby = jnp.minimum(y2, vy2[k])
                wx = jnp.maximum(rbx - ltx, 0.0)
                wy = jnp.maximum(rby - lty, 0.0)
                inter = wx * wy
                den = (ar + va[k]) - inter + 1e-10
                iou = inter / den
                if g == 0:
                    btc = iou
                    bic = jnp.zeros((_L,), jnp.int32)
                else:
                    up = iou > btc
                    btc = jnp.where(up, iou, btc)
                    bic = jnp.where(up, g, bic)
                rv = gbv[g]
                rc = gbc[g]
                up2 = iou > rv
                gbv[g] = jnp.where(up2, iou, rv)
                gbc[g] = jnp.where(up2, c, rc)
            bt[pl.ds(o, _L)] = btc
            bi[pl.ds(o, _L)] = bic
        return carry

    lax.fori_loop(0, _NCHW // 2, body, 0)

    cvacc = [jnp.full((_L,), -1.0, jnp.float32) for _ in range(_NG)]
    ciacc = [jnp.zeros((_L,), jnp.int32) for _ in range(_NG)]
    for g in range(_G):
        j, k = divmod(g, _L)
        rv = gbv[g]
        rc = gbc[g]
        gidx = base + rc * _L + lane
        m = _tree_max_f(rv, redf)
        cand = jnp.where(rv == m, gidx, _PP)
        i = _tree_min_i(cand, redi)
        cvacc[j] = jnp.where(lane == k, m, cvacc[j])
        ciacc[j] = jnp.where(lane == k, i, ciacc[j])

    for j in range(_NG):
        s = slice(j * _L, (j + 1) * _L)
        cv[s] = cvacc[j]
        ci[s] = ciacc[j]

    pltpu.sync_copy(bt, bt_h.at[pl.ds(base, _PW)])
    pltpu.sync_copy(bi, bi_h.at[pl.ds(base, _PW)])
    pltpu.sync_copy(cv, cv_h.at[pl.ds(wid * _G, _G)])
    pltpu.sync_copy(ci, ci_h.at[pl.ds(wid * _G, _G)])


@functools.partial(
    pl.kernel,
    out_type=[
        jax.ShapeDtypeStruct((_PP,), jnp.float32),
        jax.ShapeDtypeStruct((_PP,), jnp.float32),
        jax.ShapeDtypeStruct((_PP,), jnp.float32),
        jax.ShapeDtypeStruct((_PP,), jnp.float32),
        jax.ShapeDtypeStruct((_PP,), jnp.int32),
    ],
    mesh=_MESH,
    scratch_types=[
        pltpu.VMEM((_PW,), jnp.float32),
        pltpu.VMEM((_PW,), jnp.int32),
        pltpu.VMEM((_NW * _G,), jnp.float32),
        pltpu.VMEM((_NW * _G,), jnp.int32),
        pltpu.VMEM((_PW,), jnp.float32),
        pltpu.VMEM((_PW,), jnp.float32),
        pltpu.VMEM((_PW,), jnp.float32),
        pltpu.VMEM((_PW,), jnp.float32),
        pltpu.VMEM((_PW,), jnp.float32),
        pltpu.VMEM((_PW,), jnp.float32),
        pltpu.VMEM((_G,), jnp.float32),
        pltpu.VMEM((_G,), jnp.float32),
        pltpu.VMEM((_G,), jnp.float32),
        pltpu.VMEM((_G,), jnp.float32),
        pltpu.VMEM((_G,), jnp.int32),
        pltpu.VMEM((_PW,), jnp.float32),
        pltpu.VMEM((_PW,), jnp.float32),
        pltpu.VMEM((_PW,), jnp.float32),
        pltpu.VMEM((_PW,), jnp.float32),
        pltpu.VMEM((_PW,), jnp.int32),
    ],
)
def _sc_encode(bt_h, bi_h, cv_h, ci_h,
               pcx_h, pcy_h, pw_h, ph_h, plw_h, plh_h,
               bcx_h, bcy_h, lbw_h, lbh_h, glab_h,
               lx_h, ly_h, lw_h, lh_h, cls_h,
               bt, bi, cv, ci,
               pcx, pcy, pw, ph, plw, plh,
               bcx, bcy, lbw, lbh, glab,
               olx, oly, olw, olh, ocls):
    wid = lax.axis_index("s") * _NC + lax.axis_index("c")
    base = wid * _PW
    pltpu.sync_copy(bt_h.at[pl.ds(base, _PW)], bt)
    pltpu.sync_copy(bi_h.at[pl.ds(base, _PW)], bi)
    pltpu.sync_copy(cv_h, cv)
    pltpu.sync_copy(ci_h, ci)
    pltpu.sync_copy(pcx_h.at[pl.ds(base, _PW)], pcx)
    pltpu.sync_copy(pcy_h.at[pl.ds(base, _PW)], pcy)
    pltpu.sync_copy(pw_h.at[pl.ds(base, _PW)], pw)
    pltpu.sync_copy(ph_h.at[pl.ds(base, _PW)], ph)
    pltpu.sync_copy(plw_h.at[pl.ds(base, _PW)], plw)
    pltpu.sync_copy(plh_h.at[pl.ds(base, _PW)], plh)
    pltpu.sync_copy(bcx_h, bcx)
    pltpu.sync_copy(bcy_h, bcy)
    pltpu.sync_copy(lbw_h, lbw)
    pltpu.sync_copy(lbh_h, lbh)
    pltpu.sync_copy(glab_h, glab)

    lane = lax.iota(jnp.int32, _L)

    def wred(w, carry):
        rvs, ris = carry
        nv = []
        ni = []
        for j in range(_NG):
            v = cv[pl.ds(w * _G + j * _L, _L)]
            ii = ci[pl.ds(w * _G + j * _L, _L)]
            rv = rvs[j]
            ri = ris[j]
            gtm = v > rv
            nv.append(jnp.where(gtm, v, rv))
            ni.append(jnp.where(gtm, ii,
                                jnp.where(v == rv, jnp.minimum(ii, ri), ri)))
        return (tuple(nv), tuple(ni))

    rvs0 = tuple(jnp.full((_L,), -1.0, jnp.float32) for _ in range(_NG))
    ris0 = tuple(jnp.full((_L,), _PP, jnp.int32) for _ in range(_NG))
    _, ris = lax.fori_loop(0, _NW, wred, (rvs0, ris0))

    for g in range(_G):
        j, k = divmod(g, _L)
        i = ris[j][k]
        loc = i - base
        locc = jnp.clip(loc, 0, _PW - 1)
        cb = (locc // _L) * _L
        kk = locc - cb
        ksel = jnp.where(loc >= 0, jnp.where(loc < _PW, kk, -1), -1)
        row = bt[pl.ds(cb, _L)]
        rowi = bi[pl.ds(cb, _L)]
        sel = lane == ksel
        bt[pl.ds(cb, _L)] = jnp.where(sel, 2.0, row)
        bi[pl.ds(cb, _L)] = jnp.where(sel, g, rowi)

    tcx = [bcx[slice(j * _L, (j + 1) * _L)] for j in range(_NG)]
    tcy = [bcy[slice(j * _L, (j + 1) * _L)] for j in range(_NG)]
    tlw = [lbw[slice(j * _L, (j + 1) * _L)] for j in range(_NG)]
    tlh = [lbh[slice(j * _L, (j + 1) * _L)] for j in range(_NG)]
    tlb = [glab[slice(j * _L, (j + 1) * _L)] for j in range(_NG)]

    def encode(c, carry):
        o = c * _L
        btv = bt[pl.ds(o, _L)]
        biv = bi[pl.ds(o, _L)]
        gcx = jnp.full((_L,), tcx[0][0], jnp.float32)
        gcy = jnp.full((_L,), tcy[0][0], jnp.float32)
        glw = jnp.full((_L,), tlw[0][0], jnp.float32)
        glh = jnp.full((_L,), tlh[0][0], jnp.float32)
        lab = jnp.full((_L,), tlb[0][0], jnp.int32)
        for g in range(1, _G):
            j, k = divmod(g, _L)
            eq = biv == g
            gcx = jnp.where(eq, tcx[j][k], gcx)
            gcy = jnp.where(eq, tcy[j][k], gcy)
            glw = jnp.where(eq, tlw[j][k], glw)
            glh = jnp.where(eq, tlh[j][k], glh)
            lab = jnp.where(eq, tlb[j][k], lab)
        olx[pl.ds(o, _L)] = (gcx - pcx[pl.ds(o, _L)]) / pw[pl.ds(o, _L)] / 0.1
        oly[pl.ds(o, _L)] = (gcy - pcy[pl.ds(o, _L)]) / ph[pl.ds(o, _L)] / 0.1
        olw[pl.ds(o, _L)] = (glw - plw[pl.ds(o, _L)]) / 0.2
        olh[pl.ds(o, _L)] = (glh - plh[pl.ds(o, _L)]) / 0.2
        cls = jnp.where(btv < _FG, -1, lab)
        cls = jnp.where(btv < _BG, 0, cls)
        ocls[pl.ds(o, _L)] = cls
        return carry

    lax.fori_loop(0, _NCHW, encode, 0)

    pltpu.sync_copy(olx, lx_h.at[pl.ds(base, _PW)])
    pltpu.sync_copy(oly, ly_h.at[pl.ds(base, _PW)])
    pltpu.sync_copy(olw, lw_h.at[pl.ds(base, _PW)])
    pltpu.sync_copy(olh, lh_h.at[pl.ds(base, _PW)])
    pltpu.sync_copy(ocls, cls_h.at[pl.ds(base, _PW)])


def kernel(gt_boxes, labels):
    gt_boxes = gt_boxes.astype(jnp.float32)
    gx1 = gt_boxes[:, 0]
    gy1 = gt_boxes[:, 1]
    gx2 = gt_boxes[:, 2]
    gy2 = gt_boxes[:, 3]
    gar = (gx2 - gx1) * (gy2 - gy1)
    bcx = (gx1 + gx2) / 2.0
    bcy = (gy1 + gy2) / 2.0
    lbw = jnp.log(gx2 - gx1)
    lbh = jnp.log(gy2 - gy1)
    glab = (labels + 1).astype(jnp.int32)

    c = {k: jnp.asarray(v) for k, v in _PC.items()}
    bt, bi, cv, ci = _sc_match(
        c["px1"], c["py1"], c["px2"], c["py2"], c["pa"],
        gx1, gy1, gx2, gy2, gar)
    lx, ly, lw, lh, cls = _sc_encode(
        bt, bi, cv, ci,
        c["pcx"], c["pcy"], c["pw"], c["ph"], c["plw"], c["plh"],
        bcx, bcy, lbw, lbh, glab)
    loc = jnp.stack([lx, ly, lw, lh], axis=1)[:_P]
    return (loc, cls[:_P])
